# baseline (speedup 1.0000x reference)
import jax
import jax.numpy as jnp
from jax.experimental import pallas as pl
from jax.experimental.pallas import tpu as pltpu

TOP_K = 5
PROMPT_LEN = 8
PRE = (TOP_K + 1) * PROMPT_LEN + 1

CHUNK = 256
NBUF = 8
LEAD = 4


def _body(task_ref, x_ref, g_ref, ep_ref, ek_ref, cls_ref, out_ref,
          buf_in, buf_out, buf_tail, in_sem, out_sem, pf_sem):
    B, S, d = x_ref.shape
    NC = S // CHUNK
    T = B * NC

    in_descs = [None] * T
    out_descs = [None] * T
    pf_descs = []

    def start_in(u):
        slot = u % NBUF
        b, j = divmod(u, NC)
        dsc = pltpu.make_async_copy(
            x_ref.at[b, pl.ds(j * CHUNK, CHUNK), :],
            buf_in.at[slot], in_sem.at[slot])
        dsc.start()
        in_descs[u] = dsc

    def prefix(b, total):
        q = total * (1.0 / S)
        qn = q / jnp.maximum(jnp.sqrt(jnp.sum(q * q)), 1e-12)
        ek = ek_ref[...]
        kn = ek / jnp.maximum(
            jnp.sqrt(jnp.sum(ek * ek, axis=1, keepdims=True)), 1e-12)
        sim = jax.lax.dot_general(
            qn, kn, (((1,), (1,)), ((), ())),
            preferred_element_type=jnp.float32)
        tid = task_ref[0]
        dsc = pltpu.make_async_copy(
            g_ref.at[pl.ds(tid * PROMPT_LEN, PROMPT_LEN), :],
            out_ref.at[b, pl.ds(0, PROMPT_LEN), :], pf_sem)
        dsc.start()
        pf_descs.append(dsc)
        col = jax.lax.broadcasted_iota(jnp.int32, sim.shape, 1)
        for k in range(TOP_K):
            idx = jnp.argmax(sim[0])
            dsc = pltpu.make_async_copy(
                ep_ref.at[pl.ds(idx * PROMPT_LEN, PROMPT_LEN), :],
                out_ref.at[b, pl.ds((k + 1) * PROMPT_LEN, PROMPT_LEN), :],
                pf_sem)
            dsc.start()
            pf_descs.append(dsc)
            sim = jnp.where(col == idx, -jnp.inf, sim)

    for u in range(min(LEAD, T)):
        start_in(u)

    totals = [None] * B
    carry = None
    for t in range(T):
        slot = t % NBUF
        b, j = divmod(t, NC)
        in_descs[t].wait()
        data = buf_in[slot]
        totals[b] = jnp.sum(data, axis=0, keepdims=True) if j == 0 \
            else totals[b] + jnp.sum(data, axis=0, keepdims=True)
        first = cls_ref[...] if j == 0 else carry
        buf_out[slot] = jnp.concatenate([first, data[:CHUNK - 1]], axis=0)
        carry = data[CHUNK - 1:CHUNK]
        dsc = pltpu.make_async_copy(
            buf_out.at[slot],
            out_ref.at[b, pl.ds(PRE - 1 + j * CHUNK, CHUNK), :],
            out_sem.at[slot])
        dsc.start()
        out_descs[t] = dsc
        u = t + LEAD
        if u < T:
            if u >= NBUF:
                out_descs[u - NBUF].wait()
            start_in(u)
        if j == NC - 1:
            buf_tail[b, 0:1, :] = carry
            dsc = pltpu.make_async_copy(
                buf_tail.at[b, pl.ds(0, 1), :],
                out_ref.at[b, pl.ds(PRE - 1 + S, 1), :], pf_sem)
            dsc.start()
            pf_descs.append(dsc)
            prefix(b, totals[b])

    for t in range(max(0, T - NBUF), T):
        out_descs[t].wait()
    for dsc in pf_descs:
        dsc.wait()


def kernel(x, g_prompts, e_prompts, e_keys, cls_token, task_id):
    B, S, d = x.shape
    n_out = PRE + S
    g_flat = g_prompts.reshape(-1, d)
    ep_flat = e_prompts.reshape(-1, d)
    cls2 = cls_token.reshape(1, d)
    task = jnp.asarray(task_id, jnp.int32).reshape(1)
    return pl.pallas_call(
        _body,
        in_specs=[
            pl.BlockSpec(memory_space=pltpu.MemorySpace.SMEM),
            pl.BlockSpec(memory_space=pltpu.MemorySpace.HBM),
            pl.BlockSpec(memory_space=pltpu.MemorySpace.HBM),
            pl.BlockSpec(memory_space=pltpu.MemorySpace.HBM),
            pl.BlockSpec(memory_space=pltpu.MemorySpace.VMEM),
            pl.BlockSpec(memory_space=pltpu.MemorySpace.VMEM),
        ],
        out_specs=pl.BlockSpec(memory_space=pltpu.MemorySpace.HBM),
        out_shape=jax.ShapeDtypeStruct((B, n_out, d), x.dtype),
        scratch_shapes=[
            pltpu.VMEM((NBUF, CHUNK, d), jnp.float32),
            pltpu.VMEM((NBUF, CHUNK, d), jnp.float32),
            pltpu.VMEM((B, 8, d), jnp.float32),
            pltpu.SemaphoreType.DMA((NBUF,)),
            pltpu.SemaphoreType.DMA((NBUF,)),
            pltpu.SemaphoreType.DMA,
        ],
    )(task, x, g_flat, ep_flat, e_keys, cls2)
